# baseline (device time: 124165 ns/iter reference)
import jax
import jax.numpy as jnp
from jax import lax
from jax.experimental import pallas as pl
from jax.experimental.pallas import tpu as pltpu

N_DEV = 16


def kernel(x, router_W, route_idx, expert_W):
    n_tok, d_model = x.shape
    n_exp = router_W.shape[1]
    e_per, _, d_ff = expert_W.shape

    def body(x_ref, rw_ref, idx_ref, ew_ref, out_ref,
             comm_ref, send_sems, recv_sems, credit_sem):
        my = lax.axis_index("i")
        left = lax.rem(my - 1 + N_DEV, N_DEV)
        right = lax.rem(my + 1, N_DEV)

        barrier_sem = pltpu.get_barrier_semaphore()
        for nbr in (left, right):
            pl.semaphore_signal(
                barrier_sem, inc=1,
                device_id=(nbr,), device_id_type=pl.DeviceIdType.MESH,
            )
        pl.semaphore_wait(barrier_sem, 2)

        x_val = x_ref[:, :]
        scores = jnp.dot(x_val, rw_ref[:, :], preferred_element_type=jnp.float32)
        s_max = jnp.max(scores, axis=1, keepdims=True)
        p = jnp.exp(scores - s_max)
        probs = p / jnp.sum(p, axis=1, keepdims=True)

        eids = lax.broadcasted_iota(jnp.int32, (n_tok, n_exp), 1)
        mask0 = (eids == idx_ref[:, 0:1]).astype(jnp.float32)
        mask1 = (eids == idx_ref[:, 1:2]).astype(jnp.float32)
        g_raw = probs * (mask0 + mask1)
        gates = g_raw / jnp.sum(g_raw, axis=1, keepdims=True)

        def accum(origin, w_pair):
            y0 = jnp.dot(x_val, w_pair[0], preferred_element_type=jnp.float32)
            y1 = jnp.dot(x_val, w_pair[1], preferred_element_type=jnp.float32)
            g0 = jnp.sum(
                jnp.where(eids == e_per * origin, gates, 0.0),
                axis=1, keepdims=True)
            g1 = jnp.sum(
                jnp.where(eids == e_per * origin + 1, gates, 0.0),
                axis=1, keepdims=True)
            return g0 * y0 + g1 * y1

        out_ref[:, :] = accum(my, ew_ref[:, :, :])
        comm_ref[0, :, :, :] = ew_ref[:, :, :]

        for h in range(N_DEV - 1):
            s = h % 2
            r = (h + 1) % 2
            if h >= 1:
                pl.semaphore_wait(credit_sem, 1)
            rdma = pltpu.make_async_remote_copy(
                src_ref=comm_ref.at[s],
                dst_ref=comm_ref.at[r],
                send_sem=send_sems.at[s],
                recv_sem=recv_sems.at[r],
                device_id=(right,),
                device_id_type=pl.DeviceIdType.MESH,
            )
            rdma.start()
            rdma.wait()
            if h < N_DEV - 2:
                pl.semaphore_signal(
                    credit_sem, inc=1,
                    device_id=(left,), device_id_type=pl.DeviceIdType.MESH,
                )
            origin = lax.rem(my - (h + 1) + N_DEV, N_DEV)
            out_ref[:, :] += accum(origin, comm_ref[r, :, :, :])

    return pl.pallas_call(
        body,
        out_shape=jax.ShapeDtypeStruct((n_tok, d_ff), jnp.float32),
        in_specs=[
            pl.BlockSpec(memory_space=pltpu.VMEM),
            pl.BlockSpec(memory_space=pltpu.VMEM),
            pl.BlockSpec(memory_space=pltpu.VMEM),
            pl.BlockSpec(memory_space=pltpu.VMEM),
        ],
        out_specs=pl.BlockSpec(memory_space=pltpu.VMEM),
        scratch_shapes=[
            pltpu.VMEM((2, e_per, d_model, d_ff), jnp.float32),
            pltpu.SemaphoreType.DMA((2,)),
            pltpu.SemaphoreType.DMA((2,)),
            pltpu.SemaphoreType.REGULAR,
        ],
        compiler_params=pltpu.CompilerParams(collective_id=0),
    )(x, router_W, route_idx, expert_W)


# device time: 44370 ns/iter; 2.7984x vs baseline; 2.7984x over previous
import jax
import jax.numpy as jnp
from jax import lax
from jax.experimental import pallas as pl
from jax.experimental.pallas import tpu as pltpu

N_DEV = 16
HR = 8
HL = 7


def kernel(x, router_W, route_idx, expert_W):
    n_tok, d_model = x.shape
    n_exp = router_W.shape[1]
    e_per, _, d_ff = expert_W.shape

    def body(x_ref, rw_ref, idx_ref, ew_ref, out_ref,
             comm_r, comm_l, send_r, send_l, recv_r, recv_l):
        my = lax.axis_index("i")
        left = lax.rem(my - 1 + N_DEV, N_DEV)
        right = lax.rem(my + 1, N_DEV)

        barrier_sem = pltpu.get_barrier_semaphore()
        for nbr in (left, right):
            pl.semaphore_signal(
                barrier_sem, inc=1,
                device_id=(nbr,), device_id_type=pl.DeviceIdType.MESH,
            )
        pl.semaphore_wait(barrier_sem, 2)

        def mk(src, dst, ssem, rsem, dev):
            return pltpu.make_async_remote_copy(
                src_ref=src, dst_ref=dst, send_sem=ssem, recv_sem=rsem,
                device_id=(dev,), device_id_type=pl.DeviceIdType.MESH,
            )

        mk(ew_ref, comm_r.at[0], send_r.at[0], recv_r.at[0], right).start()
        mk(ew_ref, comm_l.at[0], send_l.at[0], recv_l.at[0], left).start()

        x_val = x_ref[:, :]
        scores = jnp.dot(x_val, rw_ref[:, :], preferred_element_type=jnp.float32)
        s_max = jnp.max(scores, axis=1, keepdims=True)
        p = jnp.exp(scores - s_max)
        probs = p / jnp.sum(p, axis=1, keepdims=True)

        eids = lax.broadcasted_iota(jnp.int32, (n_tok, n_exp), 1)
        mask0 = (eids == idx_ref[:, 0:1]).astype(jnp.float32)
        mask1 = (eids == idx_ref[:, 1:2]).astype(jnp.float32)
        g_raw = probs * (mask0 + mask1)
        gates = g_raw / jnp.sum(g_raw, axis=1, keepdims=True)

        def accum(origin, w_pair):
            y0 = jnp.dot(x_val, w_pair[0], preferred_element_type=jnp.float32)
            y1 = jnp.dot(x_val, w_pair[1], preferred_element_type=jnp.float32)
            g0 = jnp.sum(
                jnp.where(eids == e_per * origin, gates, 0.0),
                axis=1, keepdims=True)
            g1 = jnp.sum(
                jnp.where(eids == e_per * origin + 1, gates, 0.0),
                axis=1, keepdims=True)
            return g0 * y0 + g1 * y1

        out_ref[:, :] = accum(my, ew_ref[:, :, :])

        for h in range(HR):
            mk(comm_r.at[h], comm_r.at[h], send_r.at[h], recv_r.at[h],
               right).wait_recv()
            if h + 1 < HR:
                mk(comm_r.at[h], comm_r.at[h + 1], send_r.at[h + 1],
                   recv_r.at[h + 1], right).start()
            o = lax.rem(my - 1 - h + N_DEV, N_DEV)
            out_ref[:, :] += accum(o, comm_r[h])

            if h < HL:
                mk(comm_l.at[h], comm_l.at[h], send_l.at[h], recv_l.at[h],
                   left).wait_recv()
                if h + 1 < HL:
                    mk(comm_l.at[h], comm_l.at[h + 1], send_l.at[h + 1],
                       recv_l.at[h + 1], left).start()
                o = lax.rem(my + 1 + h, N_DEV)
                out_ref[:, :] += accum(o, comm_l[h])

        mk(ew_ref, comm_r.at[0], send_r.at[0], recv_r.at[0], right).wait_send()
        mk(ew_ref, comm_l.at[0], send_l.at[0], recv_l.at[0], left).wait_send()
        for h in range(1, HR):
            mk(comm_r.at[h - 1], comm_r.at[h], send_r.at[h], recv_r.at[h],
               right).wait_send()
        for h in range(1, HL):
            mk(comm_l.at[h - 1], comm_l.at[h], send_l.at[h], recv_l.at[h],
               left).wait_send()

    return pl.pallas_call(
        body,
        out_shape=jax.ShapeDtypeStruct((n_tok, d_ff), jnp.float32),
        in_specs=[
            pl.BlockSpec(memory_space=pltpu.VMEM),
            pl.BlockSpec(memory_space=pltpu.VMEM),
            pl.BlockSpec(memory_space=pltpu.VMEM),
            pl.BlockSpec(memory_space=pltpu.VMEM),
        ],
        out_specs=pl.BlockSpec(memory_space=pltpu.VMEM),
        scratch_shapes=[
            pltpu.VMEM((HR, e_per, d_model, d_ff), jnp.float32),
            pltpu.VMEM((HL, e_per, d_model, d_ff), jnp.float32),
            pltpu.SemaphoreType.DMA((HR,)),
            pltpu.SemaphoreType.DMA((HL,)),
            pltpu.SemaphoreType.DMA((HR,)),
            pltpu.SemaphoreType.DMA((HL,)),
        ],
        compiler_params=pltpu.CompilerParams(collective_id=0),
    )(x, router_W, route_idx, expert_W)


# device time: 43224 ns/iter; 2.8726x vs baseline; 1.0265x over previous
import jax
import jax.numpy as jnp
from jax import lax
from jax.experimental import pallas as pl
from jax.experimental.pallas import tpu as pltpu

N_DEV = 16
HR = 8
HL = 7


def kernel(x, router_W, route_idx, expert_W):
    n_tok, d_model = x.shape
    n_exp = router_W.shape[1]
    e_per, _, d_ff = expert_W.shape

    ew_merged = expert_W.transpose(1, 0, 2).reshape(d_model, e_per * d_ff)

    def body(x_ref, rw_ref, idx_ref, ew_ref, out_ref,
             comm_r, comm_l, send_r, send_l, recv_r, recv_l):
        my = lax.axis_index("i")
        left = lax.rem(my - 1 + N_DEV, N_DEV)
        right = lax.rem(my + 1, N_DEV)

        barrier_sem = pltpu.get_barrier_semaphore()
        for nbr in (left, right):
            pl.semaphore_signal(
                barrier_sem, inc=1,
                device_id=(nbr,), device_id_type=pl.DeviceIdType.MESH,
            )
        pl.semaphore_wait(barrier_sem, 2)

        def mk(src, dst, ssem, rsem, dev):
            return pltpu.make_async_remote_copy(
                src_ref=src, dst_ref=dst, send_sem=ssem, recv_sem=rsem,
                device_id=(dev,), device_id_type=pl.DeviceIdType.MESH,
            )

        mk(ew_ref, comm_r.at[0], send_r.at[0], recv_r.at[0], right).start()
        mk(ew_ref, comm_l.at[0], send_l.at[0], recv_l.at[0], left).start()

        x_val = x_ref[:, :]
        scores = jnp.dot(x_val, rw_ref[:, :], preferred_element_type=jnp.float32)
        s_max = jnp.max(scores, axis=1, keepdims=True)
        p = jnp.exp(scores - s_max)
        probs = p / jnp.sum(p, axis=1, keepdims=True)

        eids = lax.broadcasted_iota(jnp.int32, (n_tok, n_exp), 1)
        mask0 = (eids == idx_ref[:, 0:1]).astype(jnp.float32)
        mask1 = (eids == idx_ref[:, 1:2]).astype(jnp.float32)
        g_raw = probs * (mask0 + mask1)
        gates = g_raw / jnp.sum(g_raw, axis=1, keepdims=True)

        def accum(origin, w2):
            y = jnp.dot(x_val, w2, preferred_element_type=jnp.float32)
            g0 = jnp.sum(
                jnp.where(eids == e_per * origin, gates, 0.0),
                axis=1, keepdims=True)
            g1 = jnp.sum(
                jnp.where(eids == e_per * origin + 1, gates, 0.0),
                axis=1, keepdims=True)
            return g0 * y[:, :d_ff] + g1 * y[:, d_ff:]

        out_ref[:, :] = accum(my, ew_ref[:, :])

        for h in range(HR):
            mk(comm_r.at[h], comm_r.at[h], send_r.at[h], recv_r.at[h],
               right).wait_recv()
            if h + 1 < HR:
                mk(comm_r.at[h], comm_r.at[h + 1], send_r.at[h + 1],
                   recv_r.at[h + 1], right).start()
            o = lax.rem(my - 1 - h + N_DEV, N_DEV)
            out_ref[:, :] += accum(o, comm_r[h])

            if h < HL:
                mk(comm_l.at[h], comm_l.at[h], send_l.at[h], recv_l.at[h],
                   left).wait_recv()
                if h + 1 < HL:
                    mk(comm_l.at[h], comm_l.at[h + 1], send_l.at[h + 1],
                       recv_l.at[h + 1], left).start()
                o = lax.rem(my + 1 + h, N_DEV)
                out_ref[:, :] += accum(o, comm_l[h])

        mk(ew_ref, comm_r.at[0], send_r.at[0], recv_r.at[0], right).wait_send()
        mk(ew_ref, comm_l.at[0], send_l.at[0], recv_l.at[0], left).wait_send()
        for h in range(1, HR):
            mk(comm_r.at[h - 1], comm_r.at[h], send_r.at[h], recv_r.at[h],
               right).wait_send()
        for h in range(1, HL):
            mk(comm_l.at[h - 1], comm_l.at[h], send_l.at[h], recv_l.at[h],
               left).wait_send()

    return pl.pallas_call(
        body,
        out_shape=jax.ShapeDtypeStruct((n_tok, d_ff), jnp.float32),
        in_specs=[
            pl.BlockSpec(memory_space=pltpu.VMEM),
            pl.BlockSpec(memory_space=pltpu.VMEM),
            pl.BlockSpec(memory_space=pltpu.VMEM),
            pl.BlockSpec(memory_space=pltpu.VMEM),
        ],
        out_specs=pl.BlockSpec(memory_space=pltpu.VMEM),
        scratch_shapes=[
            pltpu.VMEM((HR, d_model, e_per * d_ff), jnp.float32),
            pltpu.VMEM((HL, d_model, e_per * d_ff), jnp.float32),
            pltpu.SemaphoreType.DMA((HR,)),
            pltpu.SemaphoreType.DMA((HL,)),
            pltpu.SemaphoreType.DMA((HR,)),
            pltpu.SemaphoreType.DMA((HL,)),
        ],
        compiler_params=pltpu.CompilerParams(collective_id=0),
    )(x, router_W, route_idx, ew_merged)


# device time: 23983 ns/iter; 5.1772x vs baseline; 1.8023x over previous
import jax
import jax.numpy as jnp
from jax import lax
from jax.experimental import pallas as pl
from jax.experimental.pallas import tpu as pltpu

N_DEV = 16
NZ = 4
NP = 4
S = 2
NT = 7


def kernel(x, router_W, route_idx, expert_W):
    n_tok, d_model = x.shape
    n_exp = router_W.shape[1]
    e_per, _, d_ff = expert_W.shape

    ew_bf16 = expert_W.astype(jnp.bfloat16)

    def body(x_ref, rw_ref, idx_ref, ew_ref, out_ref,
             zbel, zabv, pbuf, zs_up, zs_dn, zr_bel, zr_abv, ps, pr, dsem):
        my = lax.axis_index("i")
        zi = my // NP
        pi = lax.rem(my, NP)
        base = my - pi
        right_p = base + lax.rem(pi + 1, NP)
        left_p = base + lax.rem(pi - 1 + NP, NP)
        diag_p = base + lax.rem(pi + 2, NP)

        barrier_sem = pltpu.get_barrier_semaphore()
        for d in range(1, NZ):
            @pl.when(zi + d <= NZ - 1)
            def _():
                pl.semaphore_signal(
                    barrier_sem, inc=1, device_id=(my + NP * d,),
                    device_id_type=pl.DeviceIdType.MESH)

            @pl.when(zi >= d)
            def _():
                pl.semaphore_signal(
                    barrier_sem, inc=1, device_id=(my - NP * d,),
                    device_id_type=pl.DeviceIdType.MESH)
        for nbr in (left_p, right_p):
            pl.semaphore_signal(
                barrier_sem, inc=1, device_id=(nbr,),
                device_id_type=pl.DeviceIdType.MESH)
        pl.semaphore_wait(barrier_sem, NZ - 1 + 2)

        def mk(src, dst, ssem, rsem, dev):
            return pltpu.make_async_remote_copy(
                src_ref=src, dst_ref=dst, send_sem=ssem, recv_sem=rsem,
                device_id=(dev,), device_id_type=pl.DeviceIdType.MESH,
            )

        for d in range(1, NZ):
            @pl.when(zi + d <= NZ - 1)
            def _():
                for s in range(S):
                    mk(ew_ref.at[s], zbel.at[d - 1, s], zs_up.at[d - 1, s],
                       zr_bel.at[d - 1, s], my + NP * d).start()

            @pl.when(zi >= d)
            def _():
                for s in range(S):
                    mk(ew_ref.at[s], zabv.at[d - 1, s], zs_dn.at[d - 1, s],
                       zr_abv.at[d - 1, s], my - NP * d).start()

        for s in range(S):
            mk(ew_ref.at[s], pbuf.at[0, 0, s], ps.at[0, 0, s],
               pr.at[0, 0, s], right_p).start()
            mk(ew_ref.at[s], pbuf.at[0, 1, s], ps.at[0, 1, s],
               pr.at[0, 1, s], left_p).start()

        x_val = x_ref[:, :]
        scores = jnp.dot(x_val, rw_ref[:, :], preferred_element_type=jnp.float32)
        s_max = jnp.max(scores, axis=1, keepdims=True)
        p = jnp.exp(scores - s_max)
        probs = p / jnp.sum(p, axis=1, keepdims=True)

        eids = lax.broadcasted_iota(jnp.int32, (n_tok, n_exp), 1)
        mask0 = (eids == idx_ref[:, 0:1]).astype(jnp.float32)
        mask1 = (eids == idx_ref[:, 1:2]).astype(jnp.float32)
        g_raw = probs * (mask0 + mask1)
        gates = g_raw / jnp.sum(g_raw, axis=1, keepdims=True)

        x_bf16 = x_val.astype(jnp.bfloat16)

        def accum_sub(origin, w_s, s, first=False):
            y = jnp.dot(x_bf16, w_s, preferred_element_type=jnp.float32)
            g = jnp.sum(
                jnp.where(eids == e_per * origin + s, gates, 0.0),
                axis=1, keepdims=True)
            if first:
                out_ref[:, :] = g * y
            else:
                out_ref[:, :] += g * y

        for s in range(S):
            accum_sub(my, ew_ref[s], s, first=(s == 0))

        def zcond(t):
            if t == 0:
                return None
            d = t if t <= NZ - 1 else t - (NZ - 1)
            return (zi >= d) if t <= NZ - 1 else (zi + d <= NZ - 1)

        def zoff(t):
            if t == 0:
                return 0
            return -NP * t if t <= NZ - 1 else NP * (t - (NZ - 1))

        def process_z(d, below):
            t = d if below else (NZ - 1) + d
            cond = (zi >= d) if below else (zi + d <= NZ - 1)
            buf = zbel if below else zabv
            rsem = zr_bel if below else zr_abv
            org = my - NP * d if below else my + NP * d

            @pl.when(cond)
            def _():
                for s in range(S):
                    mk(buf.at[d - 1, s], buf.at[d - 1, s], dsem,
                       rsem.at[d - 1, s], my).wait_recv()
                    mk(buf.at[d - 1, s], pbuf.at[t, 0, s], ps.at[t, 0, s],
                       pr.at[t, 0, s], right_p).start()
                    mk(buf.at[d - 1, s], pbuf.at[t, 1, s], ps.at[t, 1, s],
                       pr.at[t, 1, s], left_p).start()
                    accum_sub(org, buf[d - 1, s], s)

        def process_plane(t):
            def go():
                for s in range(S):
                    mk(pbuf.at[t, 0, s], pbuf.at[t, 0, s], dsem,
                       pr.at[t, 0, s], my).wait_recv()
                    mk(pbuf.at[t, 0, s], pbuf.at[t, 2, s], ps.at[t, 2, s],
                       pr.at[t, 2, s], right_p).start()
                    accum_sub(left_p + zoff(t), pbuf[t, 0, s], s)
                for s in range(S):
                    mk(pbuf.at[t, 1, s], pbuf.at[t, 1, s], dsem,
                       pr.at[t, 1, s], my).wait_recv()
                    accum_sub(right_p + zoff(t), pbuf[t, 1, s], s)
            c = zcond(t)
            if c is None:
                go()
            else:
                pl.when(c)(go)

        def process_diag(t):
            def go():
                for s in range(S):
                    mk(pbuf.at[t, 2, s], pbuf.at[t, 2, s], dsem,
                       pr.at[t, 2, s], my).wait_recv()
                    accum_sub(diag_p + zoff(t), pbuf[t, 2, s], s)
            c = zcond(t)
            if c is None:
                go()
            else:
                pl.when(c)(go)

        process_z(1, below=True)
        process_z(1, below=False)
        process_plane(0)
        process_z(2, below=True)
        process_z(2, below=False)
        process_plane(1)
        process_plane(NZ)
        process_z(3, below=True)
        process_z(3, below=False)
        process_diag(0)
        process_plane(2)
        process_plane(NZ + 1)
        process_plane(3)
        process_plane(NZ + 2)
        process_diag(1)
        process_diag(NZ)
        process_diag(2)
        process_diag(NZ + 1)
        process_diag(3)
        process_diag(NZ + 2)

        for d in range(1, NZ):
            @pl.when(zi + d <= NZ - 1)
            def _():
                for s in range(S):
                    mk(ew_ref.at[s], zbel.at[d - 1, s], zs_up.at[d - 1, s],
                       dsem, my).wait_send()

            @pl.when(zi >= d)
            def _():
                for s in range(S):
                    mk(ew_ref.at[s], zabv.at[d - 1, s], zs_dn.at[d - 1, s],
                       dsem, my).wait_send()
        for t in range(NT):
            def drain(t=t):
                if t == 0:
                    def src(s):
                        return ew_ref.at[s]
                elif t <= NZ - 1:
                    def src(s, d=t):
                        return zbel.at[d - 1, s]
                else:
                    def src(s, d=t - (NZ - 1)):
                        return zabv.at[d - 1, s]
                for s in range(S):
                    for a in range(2):
                        mk(src(s), pbuf.at[t, a, s], ps.at[t, a, s],
                           dsem, my).wait_send()
                    mk(pbuf.at[t, 0, s], pbuf.at[t, 2, s], ps.at[t, 2, s],
                       dsem, my).wait_send()
            c = zcond(t)
            if c is None:
                drain()
            else:
                pl.when(c)(drain)

    return pl.pallas_call(
        body,
        out_shape=jax.ShapeDtypeStruct((n_tok, d_ff), jnp.float32),
        in_specs=[
            pl.BlockSpec(memory_space=pltpu.VMEM),
            pl.BlockSpec(memory_space=pltpu.VMEM),
            pl.BlockSpec(memory_space=pltpu.VMEM),
            pl.BlockSpec(memory_space=pltpu.VMEM),
        ],
        out_specs=pl.BlockSpec(memory_space=pltpu.VMEM),
        scratch_shapes=[
            pltpu.VMEM((NZ - 1, S, d_model, d_ff), jnp.bfloat16),
            pltpu.VMEM((NZ - 1, S, d_model, d_ff), jnp.bfloat16),
            pltpu.VMEM((NT, 3, S, d_model, d_ff), jnp.bfloat16),
            pltpu.SemaphoreType.DMA((NZ - 1, S)),
            pltpu.SemaphoreType.DMA((NZ - 1, S)),
            pltpu.SemaphoreType.DMA((NZ - 1, S)),
            pltpu.SemaphoreType.DMA((NZ - 1, S)),
            pltpu.SemaphoreType.DMA((NT, 3, S)),
            pltpu.SemaphoreType.DMA((NT, 3, S)),
            pltpu.SemaphoreType.DMA,
        ],
        compiler_params=pltpu.CompilerParams(collective_id=0),
    )(x, router_W, route_idx, ew_bf16)
